# baseline (speedup 1.0000x reference)
import functools

import jax
import jax.numpy as jnp
from jax import lax
from jax.experimental import pallas as pl
from jax.experimental.pallas import tpu as pltpu
from jax.experimental.pallas import tpu_sc as plsc

NC = 2
NS = 16
LANES = 16
CHUNK = 32


def _mm3_body(x_ref, w1_ref, w2_ref, w4_ref, k_ref, q_ref, v_ref):
    x = x_ref[...]
    k_ref[...] = jnp.dot(x, w1_ref[...], preferred_element_type=jnp.float32)
    q_ref[...] = jnp.dot(x, w2_ref[...], preferred_element_type=jnp.float32)
    v_ref[...] = jnp.dot(x, w4_ref[...], preferred_element_type=jnp.float32)


def _wrep_body(x_ref, o_ref):
    cc = lax.broadcasted_iota(jnp.int32, (128, LANES * 128), 0)
    pp = lax.broadcasted_iota(jnp.int32, (128, LANES * 128), 1)
    sel = (cc == pp // LANES).astype(jnp.float32)
    o_ref[...] = jnp.dot(x_ref[...], sel,
                         preferred_element_type=jnp.float32)


def _finish_body(x_ref, w3_ref, a0_ref, a1_ref, o_ref):
    h = jnp.dot(x_ref[...], w3_ref[...], preferred_element_type=jnp.float32)
    o_ref[...] = jnp.tanh(h + a0_ref[...] + a1_ref[...])


def _edge_sc(k, q, v, src, dst, w, zeros, cpw):
    n, d = k.shape
    rows_pt = n // NS
    epw = cpw * CHUNK
    ngrp = d // LANES
    C = CHUNK

    mesh = plsc.VectorSubcoreMesh(core_axis_name="c", subcore_axis_name="s")

    @functools.partial(
        pl.kernel,
        mesh=mesh,
        out_type=jax.ShapeDtypeStruct((NC, n, d), jnp.float32),
        scratch_types=[
            [pltpu.VMEM((C,), jnp.int32)] * 2,
            [pltpu.VMEM((C,), jnp.int32)] * 2,
            [pltpu.VMEM((C,), jnp.int32)] * 2,
            [pltpu.VMEM((C * LANES,), jnp.float32)] * 2,
            [pltpu.VMEM((C, d), jnp.float32)] * 2,
            [pltpu.VMEM((C, d), jnp.float32)] * 2,
            [pltpu.VMEM((C, d), jnp.float32)] * 2,
            [pltpu.VMEM((C, d), jnp.float32)] * 2,
            pltpu.VMEM((C * LANES,), jnp.float32),
            [pltpu.SemaphoreType.DMA] * 2,
            [pltpu.SemaphoreType.DMA] * 2,
            [pltpu.SemaphoreType.DMA] * 2,
            pltpu.VMEM_SHARED((n, d), jnp.float32),
        ],
    )
    def edge_kernel(k_hbm, q_hbm, v_hbm, src_hbm, dst_hbm, w_hbm, zero_hbm,
                    out_hbm, srcb, dstg, dsts, wb, kr, qr, vr, msg,
                    wcur, semi, semg, sems, agg_sh):
        cc = lax.axis_index("c")
        s = lax.axis_index("s")
        wid = s * NC + cc

        r0 = s * rows_pt
        pltpu.sync_copy(zero_hbm.at[pl.ds(r0, rows_pt)],
                        agg_sh.at[pl.ds(r0, rows_pt)])
        plsc.subcore_barrier()

        def idx_pairs(c, P):
            base = pl.multiple_of(wid * epw + c * C, C)
            base16 = pl.multiple_of((wid * epw + c * C) * LANES, C * LANES)
            return [(src_hbm.at[pl.ds(base, C)], srcb[P]),
                    (dst_hbm.at[pl.ds(base, C)], dstg[P]),
                    (w_hbm.at[pl.ds(base16, C * LANES)], wb[P])]

        def gather_pairs(P):
            return [(k_hbm.at[dstg[P]], kr[P]),
                    (q_hbm.at[srcb[P]], qr[P]),
                    (v_hbm.at[srcb[P]], vr[P])]

        def issue(pairs, sem):
            for a, b in pairs:
                pltpu.async_copy(a, b, sem)

        def drain(pairs, sem):
            for a, b in pairs:
                pltpu.make_async_copy(a, b, sem).wait()

        def compute(P):
            def row_body(i, carry2):
                wvec = wcur[pl.ds(i * LANES, LANES)]
                for g in range(ngrp):
                    sl = pl.ds(g * LANES, LANES)
                    t = kr[P][i, sl] + qr[P][i, sl]
                    gate = 1.0 / (1.0 + jnp.exp(-t))
                    msg[P][i, sl] = gate * vr[P][i, sl] * wvec
                return carry2

            lax.fori_loop(0, C, row_body, 0)

        def phase(c, P, *, wait_sc, issue_idx2, next_chunk):
            Q = 1 - P
            if next_chunk:
                drain(idx_pairs(c + 1, Q), semi[Q])
                issue(gather_pairs(Q), semg[Q])
            drain(gather_pairs(P), semg[P])
            if wait_sc:
                pltpu.make_async_copy(msg[P], agg_sh.at[dsts[P]],
                                      sems[P]).wait()
            for off in range(0, C, LANES):
                dsts[P][pl.ds(off, LANES)] = dstg[P][pl.ds(off, LANES)]
            for off in range(0, C * LANES, LANES):
                wcur[pl.ds(off, LANES)] = wb[P][pl.ds(off, LANES)]
            if issue_idx2:
                issue(idx_pairs(c + 2, P), semi[P])
            compute(P)
            pltpu.async_copy(msg[P], agg_sh.at[dsts[P]], sems[P], add=True)

        issue(idx_pairs(0, 0), semi[0])
        issue(idx_pairs(1, 1), semi[1])
        drain(idx_pairs(0, 0), semi[0])
        issue(gather_pairs(0), semg[0])
        phase(0, 0, wait_sc=False, issue_idx2=True, next_chunk=True)
        phase(1, 1, wait_sc=False, issue_idx2=True, next_chunk=True)

        def body(t, carry):
            phase(2 * t, 0, wait_sc=True, issue_idx2=True, next_chunk=True)
            phase(2 * t + 1, 1, wait_sc=True, issue_idx2=True,
                  next_chunk=True)
            return carry

        lax.fori_loop(1, cpw // 2 - 1, body, 0)
        phase(cpw - 2, 0, wait_sc=True, issue_idx2=False, next_chunk=True)
        phase(cpw - 1, 1, wait_sc=True, issue_idx2=False, next_chunk=False)
        for P in (0, 1):
            pltpu.make_async_copy(msg[P], agg_sh.at[dsts[P]], sems[P]).wait()

        plsc.subcore_barrier()
        pltpu.sync_copy(agg_sh.at[pl.ds(r0, rows_pt)],
                        out_hbm.at[cc, pl.ds(r0, rows_pt)])

    return edge_kernel(k, q, v, src, dst, w, zeros)


def kernel(x, edge_index, edge_weight, W1, W2, W3, W4):
    n, d = x.shape
    e = edge_weight.shape[0]
    nw = NC * NS
    cpw = -(-e // (nw * CHUNK))
    cpw = -(-cpw // 8) * 8
    cpw = max(cpw, 8)
    e_pad = nw * cpw * CHUNK
    pad = e_pad - e

    src = jnp.concatenate([edge_index[0], jnp.zeros((pad,), jnp.int32)])
    dst = jnp.concatenate([edge_index[1], jnp.zeros((pad,), jnp.int32)])
    w = jnp.concatenate([edge_weight, jnp.zeros((pad,), jnp.float32)])
    nwrow = e_pad // 128
    wblk = nwrow // 8
    w16 = pl.pallas_call(
        _wrep_body,
        grid=(8,),
        in_specs=[pl.BlockSpec((wblk, 128), lambda i: (i, 0))],
        out_specs=pl.BlockSpec((wblk, LANES * 128), lambda i: (i, 0)),
        out_shape=jax.ShapeDtypeStruct((nwrow, LANES * 128), jnp.float32),
    )(w.reshape(nwrow, 128)).reshape(-1)

    n_pad = -(-n // (NS * 8)) * (NS * 8)
    x_p = jnp.pad(x, ((0, n_pad - n), (0, 0)))
    zeros = jnp.zeros((n_pad, d), jnp.float32)

    blk = n_pad // 8 if (n_pad // 8) % 8 == 0 else NS * 8
    grid = n_pad // blk
    row_spec = pl.BlockSpec((blk, d), lambda i: (i, 0))
    w_spec = pl.BlockSpec((d, d), lambda i: (0, 0))

    k, q, v = pl.pallas_call(
        _mm3_body,
        grid=(grid,),
        in_specs=[row_spec, w_spec, w_spec, w_spec],
        out_specs=[row_spec, row_spec, row_spec],
        out_shape=[jax.ShapeDtypeStruct((n_pad, d), jnp.float32)] * 3,
    )(x_p, W1, W2, W4)

    agg = _edge_sc(k, q, v, src, dst, w16, zeros, cpw)

    out = pl.pallas_call(
        _finish_body,
        grid=(grid,),
        in_specs=[row_spec, w_spec, row_spec, row_spec],
        out_specs=row_spec,
        out_shape=jax.ShapeDtypeStruct((n_pad, d), jnp.float32),
    )(x_p, W3, agg[0], agg[1])
    return out[:n]


# speedup vs baseline: 1.6227x; 1.6227x over previous
import functools

import jax
import jax.numpy as jnp
from jax import lax
from jax.experimental import pallas as pl
from jax.experimental.pallas import tpu as pltpu
from jax.experimental.pallas import tpu_sc as plsc

NC = 2
NS = 16
LANES = 16
CHUNK = 32


def _mm3_body(x_ref, w1_ref, w2_ref, w4_ref, k_ref, q_ref, v_ref):
    x = x_ref[...]
    k_ref[...] = jnp.dot(x, w1_ref[...], preferred_element_type=jnp.float32)
    q_ref[...] = jnp.dot(x, w2_ref[...], preferred_element_type=jnp.float32)
    v_ref[...] = jnp.dot(x, w4_ref[...], preferred_element_type=jnp.float32)


def _wrep_body(x_ref, s_ref, o_ref):
    o_ref[...] = jnp.dot(x_ref[...], s_ref[...],
                         preferred_element_type=jnp.float32,
                         precision=lax.Precision.HIGHEST)


def _finish_body(x_ref, w3_ref, a0_ref, a1_ref, o_ref):
    h = jnp.dot(x_ref[...], w3_ref[...], preferred_element_type=jnp.float32)
    o_ref[...] = jnp.tanh(h + a0_ref[...] + a1_ref[...])


def _edge_sc(k, q, v, src, dst, w, zeros, cpw):
    n, d = k.shape
    rows_pt = n // NS
    epw = cpw * CHUNK
    ngrp = d // LANES
    C = CHUNK

    mesh = plsc.VectorSubcoreMesh(core_axis_name="c", subcore_axis_name="s")

    @functools.partial(
        pl.kernel,
        mesh=mesh,
        out_type=jax.ShapeDtypeStruct((NC, n, d), jnp.float32),
        scratch_types=[
            [pltpu.VMEM((C,), jnp.int32)] * 2,
            [pltpu.VMEM((C,), jnp.int32)] * 2,
            [pltpu.VMEM((C,), jnp.int32)] * 2,
            [pltpu.VMEM((C * LANES,), jnp.float32)] * 2,
            [pltpu.VMEM((C, d), jnp.float32)] * 2,
            [pltpu.VMEM((C, d), jnp.float32)] * 2,
            [pltpu.VMEM((C, d), jnp.float32)] * 2,
            [pltpu.VMEM((C, d), jnp.float32)] * 2,
            pltpu.VMEM((C * LANES,), jnp.float32),
            [pltpu.SemaphoreType.DMA] * 2,
            [pltpu.SemaphoreType.DMA] * 2,
            [pltpu.SemaphoreType.DMA] * 2,
            pltpu.VMEM_SHARED((n, d), jnp.float32),
        ],
    )
    def edge_kernel(k_hbm, q_hbm, v_hbm, src_hbm, dst_hbm, w_hbm, zero_hbm,
                    out_hbm, srcb, dstg, dsts, wb, kr, qr, vr, msg,
                    wcur, semi, semg, sems, agg_sh):
        cc = lax.axis_index("c")
        s = lax.axis_index("s")
        wid = s * NC + cc

        r0 = s * rows_pt
        pltpu.sync_copy(zero_hbm.at[pl.ds(r0, rows_pt)],
                        agg_sh.at[pl.ds(r0, rows_pt)])
        plsc.subcore_barrier()

        def idx_pairs(c, P):
            base = pl.multiple_of(wid * epw + c * C, C)
            base16 = pl.multiple_of((wid * epw + c * C) * LANES, C * LANES)
            return [(src_hbm.at[pl.ds(base, C)], srcb[P]),
                    (dst_hbm.at[pl.ds(base, C)], dstg[P]),
                    (w_hbm.at[pl.ds(base16, C * LANES)], wb[P])]

        def gather_pairs(P):
            return [(k_hbm.at[dstg[P]], kr[P]),
                    (q_hbm.at[srcb[P]], qr[P]),
                    (v_hbm.at[srcb[P]], vr[P])]

        def issue(pairs, sem):
            for a, b in pairs:
                pltpu.async_copy(a, b, sem)

        def drain(pairs, sem):
            for a, b in pairs:
                pltpu.make_async_copy(a, b, sem).wait()

        def compute(P):
            def row_body(i, carry2):
                wvec = wcur[pl.ds(i * LANES, LANES)]
                for g in range(ngrp):
                    sl = pl.ds(g * LANES, LANES)
                    t = kr[P][i, sl] + qr[P][i, sl]
                    gate = 1.0 / (1.0 + jnp.exp(-t))
                    msg[P][i, sl] = gate * vr[P][i, sl] * wvec
                return carry2

            lax.fori_loop(0, C, row_body, 0)

        def phase(c, P, *, wait_sc, issue_idx2, next_chunk):
            Q = 1 - P
            if next_chunk:
                drain(idx_pairs(c + 1, Q), semi[Q])
                issue(gather_pairs(Q), semg[Q])
            drain(gather_pairs(P), semg[P])
            if wait_sc:
                pltpu.make_async_copy(msg[P], agg_sh.at[dsts[P]],
                                      sems[P]).wait()
            for off in range(0, C, LANES):
                dsts[P][pl.ds(off, LANES)] = dstg[P][pl.ds(off, LANES)]
            for off in range(0, C * LANES, LANES):
                wcur[pl.ds(off, LANES)] = wb[P][pl.ds(off, LANES)]
            if issue_idx2:
                issue(idx_pairs(c + 2, P), semi[P])
            compute(P)
            pltpu.async_copy(msg[P], agg_sh.at[dsts[P]], sems[P], add=True)

        issue(idx_pairs(0, 0), semi[0])
        issue(idx_pairs(1, 1), semi[1])
        drain(idx_pairs(0, 0), semi[0])
        issue(gather_pairs(0), semg[0])
        phase(0, 0, wait_sc=False, issue_idx2=True, next_chunk=True)
        phase(1, 1, wait_sc=False, issue_idx2=True, next_chunk=True)

        def body(t, carry):
            phase(2 * t, 0, wait_sc=True, issue_idx2=True, next_chunk=True)
            phase(2 * t + 1, 1, wait_sc=True, issue_idx2=True,
                  next_chunk=True)
            return carry

        lax.fori_loop(1, cpw // 2 - 1, body, 0)
        phase(cpw - 2, 0, wait_sc=True, issue_idx2=False, next_chunk=True)
        phase(cpw - 1, 1, wait_sc=True, issue_idx2=False, next_chunk=False)
        for P in (0, 1):
            pltpu.make_async_copy(msg[P], agg_sh.at[dsts[P]], sems[P]).wait()

        plsc.subcore_barrier()
        pltpu.sync_copy(agg_sh.at[pl.ds(r0, rows_pt)],
                        out_hbm.at[cc, pl.ds(r0, rows_pt)])

    return edge_kernel(k, q, v, src, dst, w, zeros)


def kernel(x, edge_index, edge_weight, W1, W2, W3, W4):
    n, d = x.shape
    e = edge_weight.shape[0]
    nw = NC * NS
    cpw = -(-e // (nw * CHUNK))
    cpw += cpw % 2
    cpw = max(cpw, 4)
    e_pad = nw * cpw * CHUNK
    pad = e_pad - e

    spread = jnp.arange(pad, dtype=jnp.int32) % jnp.int32(n)
    src = jnp.concatenate([edge_index[0], spread])
    dst = jnp.concatenate([edge_index[1], spread])
    w = jnp.concatenate([edge_weight, jnp.zeros((pad,), jnp.float32)])
    nwrow = e_pad // 128
    sel = (jnp.arange(128, dtype=jnp.int32)[:, None]
           == jnp.arange(LANES * 128, dtype=jnp.int32)[None, :] // LANES
           ).astype(jnp.float32)
    w16 = pl.pallas_call(
        _wrep_body,
        grid=(1,),
        in_specs=[pl.BlockSpec((nwrow, 128), lambda i: (0, 0)),
                  pl.BlockSpec((128, LANES * 128), lambda i: (0, 0))],
        out_specs=pl.BlockSpec((nwrow, LANES * 128), lambda i: (0, 0)),
        out_shape=jax.ShapeDtypeStruct((nwrow, LANES * 128), jnp.float32),
    )(w.reshape(nwrow, 128), sel).reshape(-1)

    n_pad = -(-n // (NS * 8)) * (NS * 8)
    x_p = jnp.pad(x, ((0, n_pad - n), (0, 0)))
    zeros = jnp.zeros((n_pad, d), jnp.float32)

    blk = n_pad // 8 if (n_pad // 8) % 8 == 0 else NS * 8
    grid = n_pad // blk
    row_spec = pl.BlockSpec((blk, d), lambda i: (i, 0))
    w_spec = pl.BlockSpec((d, d), lambda i: (0, 0))

    k, q, v = pl.pallas_call(
        _mm3_body,
        grid=(grid,),
        in_specs=[row_spec, w_spec, w_spec, w_spec],
        out_specs=[row_spec, row_spec, row_spec],
        out_shape=[jax.ShapeDtypeStruct((n_pad, d), jnp.float32)] * 3,
    )(x_p, W1, W2, W4)

    agg = _edge_sc(k, q, v, src, dst, w16, zeros, cpw)

    out = pl.pallas_call(
        _finish_body,
        grid=(grid,),
        in_specs=[row_spec, w_spec, row_spec, row_spec],
        out_specs=row_spec,
        out_shape=jax.ShapeDtypeStruct((n_pad, d), jnp.float32),
    )(x_p, W3, agg[0], agg[1])
    return out[:n]
